# baseline (speedup 1.0000x reference)
import functools

import jax
import jax.numpy as jnp
from jax.experimental import pallas as pl


def _mm_body(x_ref, w_ref, b_ref, o_ref):
    o_ref[...] = (
        jnp.dot(x_ref[...], w_ref[...], preferred_element_type=jnp.float32)
        + b_ref[...]
    )


def _mm(x, w, b, blk):
    m, k = x.shape
    _, ho = w.shape
    b2 = b.reshape(1, ho)
    return pl.pallas_call(
        _mm_body,
        grid=(m // blk,),
        in_specs=[
            pl.BlockSpec((blk, k), lambda i: (i, 0)),
            pl.BlockSpec((k, ho), lambda i: (0, 0)),
            pl.BlockSpec((1, ho), lambda i: (0, 0)),
        ],
        out_specs=pl.BlockSpec((blk, ho), lambda i: (i, 0)),
        out_shape=jax.ShapeDtypeStruct((m, ho), jnp.float32),
    )(x, w, b2)


def kernel(x, a, edge_index, edge_attr, W_in, b_in, W_e, b_e, A, B, C, Dm, Em,
           bA, bB, bC, bD, bE, Wr, br):
    src = edge_index[0]
    dst = edge_index[1]
    n = x.shape[0]
    L = A.shape[0]

    xa = jnp.concatenate([x, a], axis=-1)
    h = _mm(xa, W_in, b_in, 400)
    e = _mm(edge_attr, W_e, b_e, 2000)

    for l in range(L):
        h_in, e_in = h, e
        Wn = jnp.concatenate([A[l], B[l], Dm[l], Em[l]], axis=1)
        bn = jnp.concatenate([bA[l], bB[l], bD[l], bE[l]], axis=0)
        HB = _mm(h, Wn, bn, 400)
        Ah = HB[:, 0:128]
        Bh = HB[:, 128:256]
        Dh = HB[:, 256:384]
        Eh = HB[:, 384:512]
        Ce = _mm(e, C[l], bC[l], 2000)
        e_new = Ce + Dh[src] + Eh[dst]
        sigma = jax.nn.sigmoid(e_new)
        num = jnp.zeros((n, 128), dtype=h.dtype).at[dst].add(sigma * Bh[src])
        den = jnp.zeros((n, 128), dtype=h.dtype).at[dst].add(sigma) + 1e-6
        h = h_in + jax.nn.relu(Ah + num / den)
        e = e_in + jax.nn.relu(e_new)

    out = _mm(h, Wr, br, 400)
    return out


# speedup vs baseline: 1.3052x; 1.3052x over previous
import functools

import jax
import jax.numpy as jnp
from jax import lax
from jax.experimental import pallas as pl
from jax.experimental.pallas import tpu as pltpu
from jax.experimental.pallas import tpu_sc as plsc

N = 10000
E = 320000
H = 128
HH = 64
CH = 80
ROWS = E // CH
NC = 2
NS = 16
RPT = ROWS // NS
NPT = N // NS
NBLK = 400
EBLK = 2000

_f32 = jnp.float32


def _t1_body(xa_ref, wi_ref, bi_ref, wn_ref, bn_ref,
             h_ref, ah_ref, srct_ref, dstt_ref):
    h = jnp.dot(xa_ref[...], wi_ref[...], preferred_element_type=_f32) + bi_ref[...]
    hb = jnp.dot(h, wn_ref[...], preferred_element_type=_f32) + bn_ref[...]
    h_ref[...] = h
    ah_ref[...] = hb[:, 0:128]
    bh = hb[:, 128:256]
    dh = hb[:, 256:384]
    eh = hb[:, 384:512]
    srct_ref[...] = jnp.stack([
        jnp.concatenate([dh[:, 0:64], bh[:, 0:64]], axis=1),
        jnp.concatenate([dh[:, 64:128], bh[:, 64:128]], axis=1),
    ], axis=0)
    dstt_ref[...] = eh


def _t3_body(h_ref, ah_ref, acc_ref, wn_ref, bn_ref,
             h1_ref, ah_ref_o, srct_ref, dstt_ref):
    acc = acc_ref[...]
    num = jnp.concatenate([acc[0, :, 0:64], acc[1, :, 0:64]], axis=1)
    den = jnp.concatenate([acc[0, :, 64:128], acc[1, :, 64:128]], axis=1)
    h1 = h_ref[...] + jnp.maximum(ah_ref[...] + num / (den + 1e-6), 0.0)
    hb = jnp.dot(h1, wn_ref[...], preferred_element_type=_f32) + bn_ref[...]
    h1_ref[...] = h1
    ah_ref_o[...] = hb[:, 0:128]
    bh = hb[:, 128:256]
    dh = hb[:, 256:384]
    eh = hb[:, 384:512]
    srct_ref[...] = jnp.stack([
        jnp.concatenate([dh[:, 0:64], bh[:, 0:64]], axis=1),
        jnp.concatenate([dh[:, 64:128], bh[:, 64:128]], axis=1),
    ], axis=0)
    dstt_ref[...] = eh


def _node_layer(xa, wi, bi, wn, bn, first):
    n = xa.shape[0]
    grid = n // NBLK
    kdim = xa.shape[-1]
    if first:
        body = _t1_body
        in_specs = [
            pl.BlockSpec((NBLK, kdim), lambda i: (i, 0)),
            pl.BlockSpec((kdim, 128), lambda i: (0, 0)),
            pl.BlockSpec((1, 128), lambda i: (0, 0)),
            pl.BlockSpec((128, 512), lambda i: (0, 0)),
            pl.BlockSpec((1, 512), lambda i: (0, 0)),
        ]
        args = (xa, wi, bi.reshape(1, 128), wn, bn.reshape(1, 512))
    else:
        body = _t3_body
        in_specs = [
            pl.BlockSpec((NBLK, 128), lambda i: (i, 0)),
            pl.BlockSpec((NBLK, 128), lambda i: (i, 0)),
            pl.BlockSpec((2, NBLK, 128), lambda i: (0, i, 0)),
            pl.BlockSpec((128, 512), lambda i: (0, 0)),
            pl.BlockSpec((1, 512), lambda i: (0, 0)),
        ]
        args = (xa, wi, bi, wn, bn.reshape(1, 512))
    return pl.pallas_call(
        body,
        grid=(grid,),
        in_specs=in_specs,
        out_specs=[
            pl.BlockSpec((NBLK, 128), lambda i: (i, 0)),
            pl.BlockSpec((NBLK, 128), lambda i: (i, 0)),
            pl.BlockSpec((2, NBLK, 128), lambda i: (0, i, 0)),
            pl.BlockSpec((NBLK, 128), lambda i: (i, 0)),
        ],
        out_shape=[
            jax.ShapeDtypeStruct((n, 128), _f32),
            jax.ShapeDtypeStruct((n, 128), _f32),
            jax.ShapeDtypeStruct((2, n, 128), _f32),
            jax.ShapeDtypeStruct((n, 128), _f32),
        ],
    )(*args)


def _t2_body(ea_ref, we_ref, c_ref, be_ref, bc_ref, ce_ref):
    wef = jnp.dot(we_ref[...], c_ref[...], preferred_element_type=_f32)
    cf = jnp.dot(be_ref[...], c_ref[...], preferred_element_type=_f32) + bc_ref[...]
    ce = jnp.dot(ea_ref[...], wef, preferred_element_type=_f32) + cf
    ce_ref[...] = jnp.stack([ce[:, 0:64], ce[:, 64:128]], axis=0)


def _edge_mm0(ea, we, c0, be, bc0):
    grid = E // EBLK
    return pl.pallas_call(
        _t2_body,
        grid=(grid,),
        in_specs=[
            pl.BlockSpec((EBLK, 16), lambda i: (i, 0)),
            pl.BlockSpec((16, 128), lambda i: (0, 0)),
            pl.BlockSpec((128, 128), lambda i: (0, 0)),
            pl.BlockSpec((1, 128), lambda i: (0, 0)),
            pl.BlockSpec((1, 128), lambda i: (0, 0)),
        ],
        out_specs=pl.BlockSpec((2, EBLK, 64), lambda i: (0, i, 0)),
        out_shape=jax.ShapeDtypeStruct((2, E, 64), _f32),
    )(ea, we, c0, be.reshape(1, 128), bc0.reshape(1, 128))


def _t4_body(ea_ref, r_ref, we_ref, c_ref, be_ref, bc_ref, ce_ref):
    r = r_ref[...]
    e1 = (jnp.dot(ea_ref[...], we_ref[...], preferred_element_type=_f32)
          + be_ref[...]
          + jnp.concatenate([r[0], r[1]], axis=1))
    ce = jnp.dot(e1, c_ref[...], preferred_element_type=_f32) + bc_ref[...]
    ce_ref[...] = jnp.stack([ce[:, 0:64], ce[:, 64:128]], axis=0)


def _edge_mm1(ea, r0, we, c1, be, bc1):
    grid = E // EBLK
    return pl.pallas_call(
        _t4_body,
        grid=(grid,),
        in_specs=[
            pl.BlockSpec((EBLK, 16), lambda i: (i, 0)),
            pl.BlockSpec((2, EBLK, 64), lambda i: (0, i, 0)),
            pl.BlockSpec((16, 128), lambda i: (0, 0)),
            pl.BlockSpec((128, 128), lambda i: (0, 0)),
            pl.BlockSpec((1, 128), lambda i: (0, 0)),
            pl.BlockSpec((1, 128), lambda i: (0, 0)),
        ],
        out_specs=pl.BlockSpec((2, EBLK, 64), lambda i: (0, i, 0)),
        out_shape=jax.ShapeDtypeStruct((2, E, 64), _f32),
    )(ea, r0, we, c1, be.reshape(1, 128), bc1.reshape(1, 128))


def _t5_body(h_ref, ah_ref, acc_ref, wr_ref, br_ref, out_ref):
    acc = acc_ref[...]
    num = jnp.concatenate([acc[0, :, 0:64], acc[1, :, 0:64]], axis=1)
    den = jnp.concatenate([acc[0, :, 64:128], acc[1, :, 64:128]], axis=1)
    h2 = h_ref[...] + jnp.maximum(ah_ref[...] + num / (den + 1e-6), 0.0)
    out_ref[...] = jnp.dot(h2, wr_ref[...], preferred_element_type=_f32) + br_ref[...]


def _head(h1, ah1, acc1, wr, br):
    grid = N // NBLK
    return pl.pallas_call(
        _t5_body,
        grid=(grid,),
        in_specs=[
            pl.BlockSpec((NBLK, 128), lambda i: (i, 0)),
            pl.BlockSpec((NBLK, 128), lambda i: (i, 0)),
            pl.BlockSpec((2, NBLK, 128), lambda i: (0, i, 0)),
            pl.BlockSpec((128, 1), lambda i: (0, 0)),
            pl.BlockSpec((1, 1), lambda i: (0, 0)),
        ],
        out_specs=pl.BlockSpec((NBLK, 1), lambda i: (i, 0)),
        out_shape=jax.ShapeDtypeStruct((N, 1), _f32),
    )(h1, ah1, acc1, wr, br.reshape(1, 1))


NZB = 624


ZR = 48


def _sc_body(write_r, src1d, dst1d, ce, srct, dstt, *out_and_scratch):
    if write_r:
        (acc_out, r_out, srcb, dstb, sadj,
         srcrows, dstrows, cebuf, zbuf, accsh) = out_and_scratch
    else:
        (acc_out, srcb, dstb, sadj,
         srcrows, dstrows, cebuf, zbuf, accsh) = out_and_scratch
        r_out = None
    c = lax.axis_index("c")
    s = lax.axis_index("s")
    c_n = c * N

    def zfill(i, carry):
        r = i // 8
        o = (i % 8) * 16
        zbuf[r, pl.ds(o, 16)] = jnp.zeros((16,), _f32)
        return carry
    lax.fori_loop(0, ZR * 8, zfill, 0)
    zoff = pl.multiple_of(s * NZB, 8)
    for z in range(NZB // ZR):
        pltpu.sync_copy(zbuf, accsh.at[pl.ds(zoff + z * ZR, ZR)])

    @pl.when(s == NS - 1)
    def _zero_tail():
        pltpu.sync_copy(zbuf.at[pl.ds(0, N - NS * NZB)],
                        accsh.at[pl.ds(NS * NZB, N - NS * NZB)])

    plsc.subcore_barrier()

    base = s * RPT

    def chunk(j, carry):
        eoff = pl.multiple_of((base + j) * CH, CH)
        pltpu.sync_copy(src1d.at[pl.ds(eoff, CH)], srcb)
        pltpu.sync_copy(dst1d.at[pl.ds(eoff, CH)], dstb)

        def adj(i, carry2):
            sl = pl.ds(i * 16, 16)
            sadj[sl] = srcb[sl] + c_n
            return carry2
        lax.fori_loop(0, CH // 16, adj, 0, unroll=True)

        pltpu.sync_copy(srct.at[sadj], srcrows)
        pltpu.sync_copy(dstt.at[dstb], dstrows)
        pltpu.sync_copy(ce.at[c, pl.ds(eoff, CH)], cebuf)

        def rowfn(r, carry2):
            for cc in range(HH // 16):
                o = cc * 16
                dh = srcrows[r, pl.ds(o, 16)]
                bh = srcrows[r, pl.ds(HH + o, 16)]
                eh = dstrows[r, pl.ds(c * HH + o, 16)]
                cv = cebuf[r, pl.ds(o, 16)]
                en = cv + dh + eh
                sg = 1.0 / (1.0 + jnp.exp(-en))
                srcrows[r, pl.ds(o, 16)] = sg * bh
                srcrows[r, pl.ds(HH + o, 16)] = sg
                if write_r:
                    cebuf[r, pl.ds(o, 16)] = jnp.maximum(en, 0.0)
            return carry2
        lax.fori_loop(0, CH, rowfn, 0)

        if write_r:
            pltpu.sync_copy(cebuf, r_out.at[c, pl.ds(eoff, CH)])
        pltpu.sync_copy(srcrows, accsh.at[dstb], add=True)
        return carry
    lax.fori_loop(0, RPT, chunk, 0)

    plsc.subcore_barrier()
    pltpu.sync_copy(accsh.at[pl.ds(zoff, NZB)],
                    acc_out.at[c, pl.ds(zoff, NZB)])

    @pl.when(s == NS - 1)
    def _out_tail():
        pltpu.sync_copy(accsh.at[pl.ds(NS * NZB, N - NS * NZB)],
                        acc_out.at[c, pl.ds(NS * NZB, N - NS * NZB)])


def _sc_edge_layer(src1d, dst1d, ce, srct, dstt, write_r):
    mesh = plsc.VectorSubcoreMesh(core_axis_name="c", subcore_axis_name="s",
                                  num_cores=NC, num_subcores=NS)
    out_type = [jax.ShapeDtypeStruct((2, N, 128), _f32)]
    if write_r:
        out_type.append(jax.ShapeDtypeStruct((2, E, 64), _f32))
    scratch = [
        pltpu.VMEM((CH,), jnp.int32),
        pltpu.VMEM((CH,), jnp.int32),
        pltpu.VMEM((CH,), jnp.int32),
        pltpu.VMEM((CH, 128), _f32),
        pltpu.VMEM((CH, 128), _f32),
        pltpu.VMEM((CH, 64), _f32),
        pltpu.VMEM((ZR, 128), _f32),
        pltpu.VMEM_SHARED((N, 128), _f32),
    ]
    fn = pl.kernel(
        functools.partial(_sc_body, write_r),
        out_type=out_type,
        mesh=mesh,
        scratch_types=scratch,
    )
    return fn(src1d, dst1d, ce, srct, dstt)


def kernel(x, a, edge_index, edge_attr, W_in, b_in, W_e, b_e, A, B, C, Dm, Em,
           bA, bB, bC, bD, bE, Wr, br):
    xa = jnp.concatenate([x, a], axis=-1)
    src1d = edge_index[0].astype(jnp.int32)
    dst1d = edge_index[1].astype(jnp.int32)

    wn0 = jnp.concatenate([A[0], B[0], Dm[0], Em[0]], axis=1)
    bn0 = jnp.concatenate([bA[0], bB[0], bD[0], bE[0]], axis=0)
    wn1 = jnp.concatenate([A[1], B[1], Dm[1], Em[1]], axis=1)
    bn1 = jnp.concatenate([bA[1], bB[1], bD[1], bE[1]], axis=0)

    h0, ah0, srct0, dstt0 = _node_layer(xa, W_in, b_in, wn0, bn0, first=True)
    ce0 = _edge_mm0(edge_attr, W_e, C[0], b_e, bC[0])
    acc0, r0 = _sc_edge_layer(
        src1d, dst1d, ce0,
        srct0.reshape(2 * N, 128), dstt0, True)

    h1, ah1, srct1, dstt1 = _node_layer(h0, ah0, acc0, wn1, bn1, first=False)
    ce1 = _edge_mm1(edge_attr, r0, W_e, C[1], b_e, bC[1])
    (acc1,) = _sc_edge_layer(
        src1d, dst1d, ce1,
        srct1.reshape(2 * N, 128), dstt1, False)

    return _head(h1, ah1, acc1, Wr, br)


# speedup vs baseline: 1.4514x; 1.1120x over previous
import functools

import jax
import jax.numpy as jnp
from jax import lax
from jax.experimental import pallas as pl
from jax.experimental.pallas import tpu as pltpu
from jax.experimental.pallas import tpu_sc as plsc

N = 10000
E = 320000
H = 128
HH = 64
CH = 80
ROWS = E // CH
NC = 2
NS = 16
RPT = ROWS // NS
NPT = N // NS
NBLK = 400
EBLK = 2000

_f32 = jnp.float32


def _t1_body(xa_ref, wi_ref, bi_ref, wn_ref, bn_ref,
             h_ref, ah_ref, srct_ref, dstt_ref):
    h = jnp.dot(xa_ref[...], wi_ref[...], preferred_element_type=_f32) + bi_ref[...]
    hb = jnp.dot(h, wn_ref[...], preferred_element_type=_f32) + bn_ref[...]
    h_ref[...] = h
    ah_ref[...] = hb[:, 0:128]
    bh = hb[:, 128:256]
    dh = hb[:, 256:384]
    eh = hb[:, 384:512]
    srct_ref[...] = jnp.stack([
        jnp.concatenate([dh[:, 0:64], bh[:, 0:64]], axis=1),
        jnp.concatenate([dh[:, 64:128], bh[:, 64:128]], axis=1),
    ], axis=0)
    dstt_ref[...] = eh


def _t3_body(h_ref, ah_ref, acc_ref, wn_ref, bn_ref,
             h1_ref, ah_ref_o, srct_ref, dstt_ref):
    acc = acc_ref[...]
    num = jnp.concatenate([acc[0, :, 0:64], acc[1, :, 0:64]], axis=1)
    den = jnp.concatenate([acc[0, :, 64:128], acc[1, :, 64:128]], axis=1)
    h1 = h_ref[...] + jnp.maximum(ah_ref[...] + num / (den + 1e-6), 0.0)
    hb = jnp.dot(h1, wn_ref[...], preferred_element_type=_f32) + bn_ref[...]
    h1_ref[...] = h1
    ah_ref_o[...] = hb[:, 0:128]
    bh = hb[:, 128:256]
    dh = hb[:, 256:384]
    eh = hb[:, 384:512]
    srct_ref[...] = jnp.stack([
        jnp.concatenate([dh[:, 0:64], bh[:, 0:64]], axis=1),
        jnp.concatenate([dh[:, 64:128], bh[:, 64:128]], axis=1),
    ], axis=0)
    dstt_ref[...] = eh


def _node_layer(xa, wi, bi, wn, bn, first):
    n = xa.shape[0]
    grid = n // NBLK
    kdim = xa.shape[-1]
    if first:
        body = _t1_body
        in_specs = [
            pl.BlockSpec((NBLK, kdim), lambda i: (i, 0)),
            pl.BlockSpec((kdim, 128), lambda i: (0, 0)),
            pl.BlockSpec((1, 128), lambda i: (0, 0)),
            pl.BlockSpec((128, 512), lambda i: (0, 0)),
            pl.BlockSpec((1, 512), lambda i: (0, 0)),
        ]
        args = (xa, wi, bi.reshape(1, 128), wn, bn.reshape(1, 512))
    else:
        body = _t3_body
        in_specs = [
            pl.BlockSpec((NBLK, 128), lambda i: (i, 0)),
            pl.BlockSpec((NBLK, 128), lambda i: (i, 0)),
            pl.BlockSpec((2, NBLK, 128), lambda i: (0, i, 0)),
            pl.BlockSpec((128, 512), lambda i: (0, 0)),
            pl.BlockSpec((1, 512), lambda i: (0, 0)),
        ]
        args = (xa, wi, bi, wn, bn.reshape(1, 512))
    return pl.pallas_call(
        body,
        grid=(grid,),
        in_specs=in_specs,
        out_specs=[
            pl.BlockSpec((NBLK, 128), lambda i: (i, 0)),
            pl.BlockSpec((NBLK, 128), lambda i: (i, 0)),
            pl.BlockSpec((2, NBLK, 128), lambda i: (0, i, 0)),
            pl.BlockSpec((NBLK, 128), lambda i: (i, 0)),
        ],
        out_shape=[
            jax.ShapeDtypeStruct((n, 128), _f32),
            jax.ShapeDtypeStruct((n, 128), _f32),
            jax.ShapeDtypeStruct((2, n, 128), _f32),
            jax.ShapeDtypeStruct((n, 128), _f32),
        ],
    )(*args)


def _t2_body(ea_ref, we_ref, c_ref, be_ref, bc_ref, ce_ref):
    wef = jnp.dot(we_ref[...], c_ref[...], preferred_element_type=_f32)
    cf = jnp.dot(be_ref[...], c_ref[...], preferred_element_type=_f32) + bc_ref[...]
    ce = jnp.dot(ea_ref[...], wef, preferred_element_type=_f32) + cf
    ce_ref[...] = jnp.stack([ce[:, 0:64], ce[:, 64:128]], axis=0)


def _edge_mm0(ea, we, c0, be, bc0):
    grid = E // EBLK
    return pl.pallas_call(
        _t2_body,
        grid=(grid,),
        in_specs=[
            pl.BlockSpec((EBLK, 16), lambda i: (i, 0)),
            pl.BlockSpec((16, 128), lambda i: (0, 0)),
            pl.BlockSpec((128, 128), lambda i: (0, 0)),
            pl.BlockSpec((1, 128), lambda i: (0, 0)),
            pl.BlockSpec((1, 128), lambda i: (0, 0)),
        ],
        out_specs=pl.BlockSpec((2, EBLK, 64), lambda i: (0, i, 0)),
        out_shape=jax.ShapeDtypeStruct((2, E, 64), _f32),
    )(ea, we, c0, be.reshape(1, 128), bc0.reshape(1, 128))


def _t4_body(ea_ref, r_ref, we_ref, c_ref, be_ref, bc_ref, ce_ref):
    r = r_ref[...]
    e1 = (jnp.dot(ea_ref[...], we_ref[...], preferred_element_type=_f32)
          + be_ref[...]
          + jnp.concatenate([r[0], r[1]], axis=1))
    ce = jnp.dot(e1, c_ref[...], preferred_element_type=_f32) + bc_ref[...]
    ce_ref[...] = jnp.stack([ce[:, 0:64], ce[:, 64:128]], axis=0)


def _edge_mm1(ea, r0, we, c1, be, bc1):
    grid = E // EBLK
    return pl.pallas_call(
        _t4_body,
        grid=(grid,),
        in_specs=[
            pl.BlockSpec((EBLK, 16), lambda i: (i, 0)),
            pl.BlockSpec((2, EBLK, 64), lambda i: (0, i, 0)),
            pl.BlockSpec((16, 128), lambda i: (0, 0)),
            pl.BlockSpec((128, 128), lambda i: (0, 0)),
            pl.BlockSpec((1, 128), lambda i: (0, 0)),
            pl.BlockSpec((1, 128), lambda i: (0, 0)),
        ],
        out_specs=pl.BlockSpec((2, EBLK, 64), lambda i: (0, i, 0)),
        out_shape=jax.ShapeDtypeStruct((2, E, 64), _f32),
    )(ea, r0, we, c1, be.reshape(1, 128), bc1.reshape(1, 128))


def _t5_body(h_ref, ah_ref, acc_ref, wr_ref, br_ref, out_ref):
    acc = acc_ref[...]
    num = jnp.concatenate([acc[0, :, 0:64], acc[1, :, 0:64]], axis=1)
    den = jnp.concatenate([acc[0, :, 64:128], acc[1, :, 64:128]], axis=1)
    h2 = h_ref[...] + jnp.maximum(ah_ref[...] + num / (den + 1e-6), 0.0)
    out_ref[...] = jnp.dot(h2, wr_ref[...], preferred_element_type=_f32) + br_ref[...]


def _head(h1, ah1, acc1, wr, br):
    grid = N // NBLK
    return pl.pallas_call(
        _t5_body,
        grid=(grid,),
        in_specs=[
            pl.BlockSpec((NBLK, 128), lambda i: (i, 0)),
            pl.BlockSpec((NBLK, 128), lambda i: (i, 0)),
            pl.BlockSpec((2, NBLK, 128), lambda i: (0, i, 0)),
            pl.BlockSpec((128, 1), lambda i: (0, 0)),
            pl.BlockSpec((1, 1), lambda i: (0, 0)),
        ],
        out_specs=pl.BlockSpec((NBLK, 1), lambda i: (i, 0)),
        out_shape=jax.ShapeDtypeStruct((N, 1), _f32),
    )(h1, ah1, acc1, wr, br.reshape(1, 1))


NZB = 624


ZR = 48


def _sc_body(write_r, src1d, dst1d, ce, srct, dstt, *out_and_scratch):
    if write_r:
        (acc_out, r_out, srcb, dstb, sadj,
         srcrows, dstrows, cebuf, zbuf, accsh, gsem) = out_and_scratch
    else:
        (acc_out, srcb, dstb, sadj,
         srcrows, dstrows, cebuf, zbuf, accsh, gsem) = out_and_scratch
        r_out = None
    c = lax.axis_index("c")
    s = lax.axis_index("s")
    c_n = c * N

    def zfill(i, carry):
        r = i // 8
        o = (i % 8) * 16
        zbuf[r, pl.ds(o, 16)] = jnp.zeros((16,), _f32)
        return carry
    lax.fori_loop(0, ZR * 8, zfill, 0)
    zoff = pl.multiple_of(s * NZB, 8)
    for z in range(NZB // ZR):
        pltpu.sync_copy(zbuf, accsh.at[pl.ds(zoff + z * ZR, ZR)])

    @pl.when(s == NS - 1)
    def _zero_tail():
        pltpu.sync_copy(zbuf.at[pl.ds(0, N - NS * NZB)],
                        accsh.at[pl.ds(NS * NZB, N - NS * NZB)])

    plsc.subcore_barrier()

    base = s * RPT

    def chunk(j, carry):
        eoff = pl.multiple_of((base + j) * CH, CH)
        pltpu.sync_copy(src1d.at[pl.ds(eoff, CH)], srcb)
        pltpu.sync_copy(dst1d.at[pl.ds(eoff, CH)], dstb)

        def adj(i, carry2):
            sl = pl.ds(i * 16, 16)
            sadj[sl] = srcb[sl] + c_n
            return carry2
        lax.fori_loop(0, CH // 16, adj, 0, unroll=True)

        d1 = pltpu.async_copy(srct.at[sadj], srcrows, gsem)
        d2 = pltpu.async_copy(dstt.at[dstb], dstrows, gsem)
        d3 = pltpu.async_copy(ce.at[c, pl.ds(eoff, CH)], cebuf, gsem)
        d1.wait()
        d2.wait()
        d3.wait()

        def rowfn(r, carry2):
            for cc in range(HH // 16):
                o = cc * 16
                dh = srcrows[r, pl.ds(o, 16)]
                bh = srcrows[r, pl.ds(HH + o, 16)]
                eh = dstrows[r, pl.ds(c * HH + o, 16)]
                cv = cebuf[r, pl.ds(o, 16)]
                en = cv + dh + eh
                sg = 1.0 / (1.0 + jnp.exp(-en))
                srcrows[r, pl.ds(o, 16)] = sg * bh
                srcrows[r, pl.ds(HH + o, 16)] = sg
                if write_r:
                    cebuf[r, pl.ds(o, 16)] = jnp.maximum(en, 0.0)
            return carry2
        lax.fori_loop(0, CH, rowfn, 0)

        if write_r:
            pltpu.sync_copy(cebuf, r_out.at[c, pl.ds(eoff, CH)])
        pltpu.sync_copy(srcrows, accsh.at[dstb], add=True)
        return carry
    lax.fori_loop(0, RPT, chunk, 0)

    plsc.subcore_barrier()
    pltpu.sync_copy(accsh.at[pl.ds(zoff, NZB)],
                    acc_out.at[c, pl.ds(zoff, NZB)])

    @pl.when(s == NS - 1)
    def _out_tail():
        pltpu.sync_copy(accsh.at[pl.ds(NS * NZB, N - NS * NZB)],
                        acc_out.at[c, pl.ds(NS * NZB, N - NS * NZB)])


def _sc_edge_layer(src1d, dst1d, ce, srct, dstt, write_r):
    mesh = plsc.VectorSubcoreMesh(core_axis_name="c", subcore_axis_name="s",
                                  num_cores=NC, num_subcores=NS)
    out_type = [jax.ShapeDtypeStruct((2, N, 128), _f32)]
    if write_r:
        out_type.append(jax.ShapeDtypeStruct((2, E, 64), _f32))
    scratch = [
        pltpu.VMEM((CH,), jnp.int32),
        pltpu.VMEM((CH,), jnp.int32),
        pltpu.VMEM((CH,), jnp.int32),
        pltpu.VMEM((CH, 128), _f32),
        pltpu.VMEM((CH, 128), _f32),
        pltpu.VMEM((CH, 64), _f32),
        pltpu.VMEM((ZR, 128), _f32),
        pltpu.VMEM_SHARED((N, 128), _f32),
        pltpu.SemaphoreType.DMA,
    ]
    fn = pl.kernel(
        functools.partial(_sc_body, write_r),
        out_type=out_type,
        mesh=mesh,
        scratch_types=scratch,
    )
    return fn(src1d, dst1d, ce, srct, dstt)


def kernel(x, a, edge_index, edge_attr, W_in, b_in, W_e, b_e, A, B, C, Dm, Em,
           bA, bB, bC, bD, bE, Wr, br):
    xa = jnp.concatenate([x, a], axis=-1)
    src1d = edge_index[0].astype(jnp.int32)
    dst1d = edge_index[1].astype(jnp.int32)

    wn0 = jnp.concatenate([A[0], B[0], Dm[0], Em[0]], axis=1)
    bn0 = jnp.concatenate([bA[0], bB[0], bD[0], bE[0]], axis=0)
    wn1 = jnp.concatenate([A[1], B[1], Dm[1], Em[1]], axis=1)
    bn1 = jnp.concatenate([bA[1], bB[1], bD[1], bE[1]], axis=0)

    h0, ah0, srct0, dstt0 = _node_layer(xa, W_in, b_in, wn0, bn0, first=True)
    ce0 = _edge_mm0(edge_attr, W_e, C[0], b_e, bC[0])
    acc0, r0 = _sc_edge_layer(
        src1d, dst1d, ce0,
        srct0.reshape(2 * N, 128), dstt0, True)

    h1, ah1, srct1, dstt1 = _node_layer(h0, ah0, acc0, wn1, bn1, first=False)
    ce1 = _edge_mm1(edge_attr, r0, W_e, C[1], b_e, bC[1])
    (acc1,) = _sc_edge_layer(
        src1d, dst1d, ce1,
        srct1.reshape(2 * N, 128), dstt1, False)

    return _head(h1, ah1, acc1, Wr, br)


# speedup vs baseline: 1.5217x; 1.0484x over previous
import functools

import jax
import jax.numpy as jnp
from jax import lax
from jax.experimental import pallas as pl
from jax.experimental.pallas import tpu as pltpu
from jax.experimental.pallas import tpu_sc as plsc

N = 10000
E = 320000
H = 128
HH = 64
CH = 80
ROWS = E // CH
NC = 2
NS = 16
RPT = ROWS // NS
NPT = N // NS
NBLK = 400
EBLK = 2000

_f32 = jnp.float32


def _t1_body(xa_ref, wi_ref, bi_ref, wn_ref, bn_ref,
             h_ref, ah_ref, srct_ref, dstt_ref):
    h = jnp.dot(xa_ref[...], wi_ref[...], preferred_element_type=_f32) + bi_ref[...]
    hb = jnp.dot(h, wn_ref[...], preferred_element_type=_f32) + bn_ref[...]
    h_ref[...] = h
    ah_ref[...] = hb[:, 0:128]
    bh = hb[:, 128:256]
    dh = hb[:, 256:384]
    eh = hb[:, 384:512]
    srct_ref[...] = jnp.stack([
        jnp.concatenate([dh[:, 0:64], bh[:, 0:64]], axis=1),
        jnp.concatenate([dh[:, 64:128], bh[:, 64:128]], axis=1),
    ], axis=0)
    dstt_ref[...] = eh


def _t3_body(h_ref, ah_ref, acc_ref, wn_ref, bn_ref,
             h1_ref, ah_ref_o, srct_ref, dstt_ref):
    acc = acc_ref[...]
    num = jnp.concatenate([acc[0, :, 0:64], acc[1, :, 0:64]], axis=1)
    den = jnp.concatenate([acc[0, :, 64:128], acc[1, :, 64:128]], axis=1)
    h1 = h_ref[...] + jnp.maximum(ah_ref[...] + num / (den + 1e-6), 0.0)
    hb = jnp.dot(h1, wn_ref[...], preferred_element_type=_f32) + bn_ref[...]
    h1_ref[...] = h1
    ah_ref_o[...] = hb[:, 0:128]
    bh = hb[:, 128:256]
    dh = hb[:, 256:384]
    eh = hb[:, 384:512]
    srct_ref[...] = jnp.stack([
        jnp.concatenate([dh[:, 0:64], bh[:, 0:64]], axis=1),
        jnp.concatenate([dh[:, 64:128], bh[:, 64:128]], axis=1),
    ], axis=0)
    dstt_ref[...] = eh


def _node_layer(xa, wi, bi, wn, bn, first):
    n = xa.shape[0]
    grid = n // NBLK
    kdim = xa.shape[-1]
    if first:
        body = _t1_body
        in_specs = [
            pl.BlockSpec((NBLK, kdim), lambda i: (i, 0)),
            pl.BlockSpec((kdim, 128), lambda i: (0, 0)),
            pl.BlockSpec((1, 128), lambda i: (0, 0)),
            pl.BlockSpec((128, 512), lambda i: (0, 0)),
            pl.BlockSpec((1, 512), lambda i: (0, 0)),
        ]
        args = (xa, wi, bi.reshape(1, 128), wn, bn.reshape(1, 512))
    else:
        body = _t3_body
        in_specs = [
            pl.BlockSpec((NBLK, 128), lambda i: (i, 0)),
            pl.BlockSpec((NBLK, 128), lambda i: (i, 0)),
            pl.BlockSpec((2, NBLK, 128), lambda i: (0, i, 0)),
            pl.BlockSpec((128, 512), lambda i: (0, 0)),
            pl.BlockSpec((1, 512), lambda i: (0, 0)),
        ]
        args = (xa, wi, bi, wn, bn.reshape(1, 512))
    return pl.pallas_call(
        body,
        grid=(grid,),
        in_specs=in_specs,
        out_specs=[
            pl.BlockSpec((NBLK, 128), lambda i: (i, 0)),
            pl.BlockSpec((NBLK, 128), lambda i: (i, 0)),
            pl.BlockSpec((2, NBLK, 128), lambda i: (0, i, 0)),
            pl.BlockSpec((NBLK, 128), lambda i: (i, 0)),
        ],
        out_shape=[
            jax.ShapeDtypeStruct((n, 128), _f32),
            jax.ShapeDtypeStruct((n, 128), _f32),
            jax.ShapeDtypeStruct((2, n, 128), _f32),
            jax.ShapeDtypeStruct((n, 128), _f32),
        ],
    )(*args)


def _t2_body(ea_ref, we_ref, c_ref, be_ref, bc_ref, ce_ref):
    wef = jnp.dot(we_ref[...], c_ref[...], preferred_element_type=_f32)
    cf = jnp.dot(be_ref[...], c_ref[...], preferred_element_type=_f32) + bc_ref[...]
    ce = jnp.dot(ea_ref[...], wef, preferred_element_type=_f32) + cf
    ce_ref[...] = jnp.stack([ce[:, 0:64], ce[:, 64:128]], axis=0)


def _edge_mm0(ea, we, c0, be, bc0):
    grid = E // EBLK
    return pl.pallas_call(
        _t2_body,
        grid=(grid,),
        in_specs=[
            pl.BlockSpec((EBLK, 16), lambda i: (i, 0)),
            pl.BlockSpec((16, 128), lambda i: (0, 0)),
            pl.BlockSpec((128, 128), lambda i: (0, 0)),
            pl.BlockSpec((1, 128), lambda i: (0, 0)),
            pl.BlockSpec((1, 128), lambda i: (0, 0)),
        ],
        out_specs=pl.BlockSpec((2, EBLK, 64), lambda i: (0, i, 0)),
        out_shape=jax.ShapeDtypeStruct((2, E, 64), _f32),
    )(ea, we, c0, be.reshape(1, 128), bc0.reshape(1, 128))


def _t4_body(ea_ref, r_ref, we_ref, c_ref, be_ref, bc_ref, ce_ref):
    r = r_ref[...]
    e1 = (jnp.dot(ea_ref[...], we_ref[...], preferred_element_type=_f32)
          + be_ref[...]
          + jnp.concatenate([r[0], r[1]], axis=1))
    ce = jnp.dot(e1, c_ref[...], preferred_element_type=_f32) + bc_ref[...]
    ce_ref[...] = jnp.stack([ce[:, 0:64], ce[:, 64:128]], axis=0)


def _edge_mm1(ea, r0, we, c1, be, bc1):
    grid = E // EBLK
    return pl.pallas_call(
        _t4_body,
        grid=(grid,),
        in_specs=[
            pl.BlockSpec((EBLK, 16), lambda i: (i, 0)),
            pl.BlockSpec((2, EBLK, 64), lambda i: (0, i, 0)),
            pl.BlockSpec((16, 128), lambda i: (0, 0)),
            pl.BlockSpec((128, 128), lambda i: (0, 0)),
            pl.BlockSpec((1, 128), lambda i: (0, 0)),
            pl.BlockSpec((1, 128), lambda i: (0, 0)),
        ],
        out_specs=pl.BlockSpec((2, EBLK, 64), lambda i: (0, i, 0)),
        out_shape=jax.ShapeDtypeStruct((2, E, 64), _f32),
    )(ea, r0, we, c1, be.reshape(1, 128), bc1.reshape(1, 128))


def _t5_body(h_ref, ah_ref, acc_ref, wr_ref, br_ref, out_ref):
    acc = acc_ref[...]
    num = jnp.concatenate([acc[0, :, 0:64], acc[1, :, 0:64]], axis=1)
    den = jnp.concatenate([acc[0, :, 64:128], acc[1, :, 64:128]], axis=1)
    h2 = h_ref[...] + jnp.maximum(ah_ref[...] + num / (den + 1e-6), 0.0)
    out_ref[...] = jnp.dot(h2, wr_ref[...], preferred_element_type=_f32) + br_ref[...]


def _head(h1, ah1, acc1, wr, br):
    grid = N // NBLK
    return pl.pallas_call(
        _t5_body,
        grid=(grid,),
        in_specs=[
            pl.BlockSpec((NBLK, 128), lambda i: (i, 0)),
            pl.BlockSpec((NBLK, 128), lambda i: (i, 0)),
            pl.BlockSpec((2, NBLK, 128), lambda i: (0, i, 0)),
            pl.BlockSpec((128, 1), lambda i: (0, 0)),
            pl.BlockSpec((1, 1), lambda i: (0, 0)),
        ],
        out_specs=pl.BlockSpec((NBLK, 1), lambda i: (i, 0)),
        out_shape=jax.ShapeDtypeStruct((N, 1), _f32),
    )(h1, ah1, acc1, wr, br.reshape(1, 1))


NZB = 624


ZR = 48


def _sc_body(write_r, src1d, dst1d, ce, srct, dstt, *out_and_scratch):
    if write_r:
        (acc_out, r_out, srcb, dstb, sadj,
         srcrows, dstrows, cebuf, zbuf, accsh, gsem) = out_and_scratch
    else:
        (acc_out, srcb, dstb, sadj,
         srcrows, dstrows, cebuf, zbuf, accsh, gsem) = out_and_scratch
        r_out = None
    c = lax.axis_index("c")
    s = lax.axis_index("s")
    c_n = c * N

    def zfill(i, carry):
        r = i // 8
        o = (i % 8) * 16
        zbuf[r, pl.ds(o, 16)] = jnp.zeros((16,), _f32)
        return carry
    lax.fori_loop(0, ZR * 8, zfill, 0)
    zoff = pl.multiple_of(s * NZB, 8)
    for z in range(NZB // ZR):
        pltpu.sync_copy(zbuf, accsh.at[pl.ds(zoff + z * ZR, ZR)])

    @pl.when(s == NS - 1)
    def _zero_tail():
        pltpu.sync_copy(zbuf.at[pl.ds(0, N - NS * NZB)],
                        accsh.at[pl.ds(NS * NZB, N - NS * NZB)])

    plsc.subcore_barrier()

    base = s * RPT

    def chunk(j, carry):
        eoff = pl.multiple_of((base + j) * CH, CH)
        i1 = pltpu.async_copy(src1d.at[pl.ds(eoff, CH)], srcb, gsem)
        i2 = pltpu.async_copy(dst1d.at[pl.ds(eoff, CH)], dstb, gsem)
        i1.wait()
        i2.wait()

        def adj(i, carry2):
            sl = pl.ds(i * 16, 16)
            sadj[sl] = srcb[sl] + c_n
            return carry2
        lax.fori_loop(0, CH // 16, adj, 0, unroll=True)

        d1 = pltpu.async_copy(srct.at[sadj], srcrows, gsem)
        d2 = pltpu.async_copy(dstt.at[dstb], dstrows, gsem)
        d3 = pltpu.async_copy(ce.at[c, pl.ds(eoff, CH)], cebuf, gsem)
        d1.wait()
        d2.wait()
        d3.wait()

        def rowfn(r, carry2):
            for cc in range(HH // 16):
                o = cc * 16
                dh = srcrows[r, pl.ds(o, 16)]
                bh = srcrows[r, pl.ds(HH + o, 16)]
                eh = dstrows[r, pl.ds(c * HH + o, 16)]
                cv = cebuf[r, pl.ds(o, 16)]
                en = cv + dh + eh
                sg = 1.0 / (1.0 + jnp.exp(-en))
                srcrows[r, pl.ds(o, 16)] = sg * bh
                srcrows[r, pl.ds(HH + o, 16)] = sg
                if write_r:
                    cebuf[r, pl.ds(o, 16)] = jnp.maximum(en, 0.0)
            return carry2
        lax.fori_loop(0, CH, rowfn, 0)

        if write_r:
            rd = pltpu.async_copy(cebuf, r_out.at[c, pl.ds(eoff, CH)], gsem)
        pltpu.sync_copy(srcrows, accsh.at[dstb], add=True)
        if write_r:
            rd.wait()
        return carry
    lax.fori_loop(0, RPT, chunk, 0)

    plsc.subcore_barrier()
    pltpu.sync_copy(accsh.at[pl.ds(zoff, NZB)],
                    acc_out.at[c, pl.ds(zoff, NZB)])

    @pl.when(s == NS - 1)
    def _out_tail():
        pltpu.sync_copy(accsh.at[pl.ds(NS * NZB, N - NS * NZB)],
                        acc_out.at[c, pl.ds(NS * NZB, N - NS * NZB)])


def _sc_edge_layer(src1d, dst1d, ce, srct, dstt, write_r):
    mesh = plsc.VectorSubcoreMesh(core_axis_name="c", subcore_axis_name="s",
                                  num_cores=NC, num_subcores=NS)
    out_type = [jax.ShapeDtypeStruct((2, N, 128), _f32)]
    if write_r:
        out_type.append(jax.ShapeDtypeStruct((2, E, 64), _f32))
    scratch = [
        pltpu.VMEM((CH,), jnp.int32),
        pltpu.VMEM((CH,), jnp.int32),
        pltpu.VMEM((CH,), jnp.int32),
        pltpu.VMEM((CH, 128), _f32),
        pltpu.VMEM((CH, 128), _f32),
        pltpu.VMEM((CH, 64), _f32),
        pltpu.VMEM((ZR, 128), _f32),
        pltpu.VMEM_SHARED((N, 128), _f32),
        pltpu.SemaphoreType.DMA,
    ]
    fn = pl.kernel(
        functools.partial(_sc_body, write_r),
        out_type=out_type,
        mesh=mesh,
        scratch_types=scratch,
    )
    return fn(src1d, dst1d, ce, srct, dstt)


def kernel(x, a, edge_index, edge_attr, W_in, b_in, W_e, b_e, A, B, C, Dm, Em,
           bA, bB, bC, bD, bE, Wr, br):
    xa = jnp.concatenate([x, a], axis=-1)
    src1d = edge_index[0].astype(jnp.int32)
    dst1d = edge_index[1].astype(jnp.int32)

    wn0 = jnp.concatenate([A[0], B[0], Dm[0], Em[0]], axis=1)
    bn0 = jnp.concatenate([bA[0], bB[0], bD[0], bE[0]], axis=0)
    wn1 = jnp.concatenate([A[1], B[1], Dm[1], Em[1]], axis=1)
    bn1 = jnp.concatenate([bA[1], bB[1], bD[1], bE[1]], axis=0)

    h0, ah0, srct0, dstt0 = _node_layer(xa, W_in, b_in, wn0, bn0, first=True)
    ce0 = _edge_mm0(edge_attr, W_e, C[0], b_e, bC[0])
    acc0, r0 = _sc_edge_layer(
        src1d, dst1d, ce0,
        srct0.reshape(2 * N, 128), dstt0, True)

    h1, ah1, srct1, dstt1 = _node_layer(h0, ah0, acc0, wn1, bn1, first=False)
    ce1 = _edge_mm1(edge_attr, r0, W_e, C[1], b_e, bC[1])
    (acc1,) = _sc_edge_layer(
        src1d, dst1d, ce1,
        srct1.reshape(2 * N, 128), dstt1, False)

    return _head(h1, ah1, acc1, Wr, br)
